# baseline (speedup 1.0000x reference)
import jax
import jax.numpy as jnp
from jax import lax
from jax.experimental import pallas as pl

_TB = 512
_E = 16
_NEG_INF = float("-inf")


def _body(w_ref, hs_ref, wout_ref, eout_ref):
    lg = jax.lax.dot_general(
        w_ref[...], hs_ref[...], (((1,), (1,)), ((), ())),
        preferred_element_type=jnp.float32,
    )
    row = lax.broadcasted_iota(jnp.int32, lg.shape, 0)
    m1 = jnp.max(lg, axis=0, keepdims=True)
    i1 = jnp.min(jnp.where(lg == m1, row, _E), axis=0, keepdims=True)
    masked = jnp.where(row == i1, _NEG_INF, lg)
    m2 = jnp.max(masked, axis=0, keepdims=True)
    i2 = jnp.min(jnp.where(masked == m2, row, _E), axis=0, keepdims=True)
    t = jnp.exp(m2 - m1)
    denom = 1.0 + t
    wout_ref[...] = jnp.concatenate([1.0 / denom, t / denom], axis=0)
    eout_ref[...] = jnp.concatenate([i1, i2], axis=0)


@jax.jit
def kernel(hidden_states, W):
    hs = hidden_states.reshape(-1, hidden_states.shape[-1])
    T, d = hs.shape
    w2t, e2t = pl.pallas_call(
        _body,
        grid=(T // _TB,),
        in_specs=[
            pl.BlockSpec((_E, d), lambda i: (0, 0)),
            pl.BlockSpec((_TB, d), lambda i: (i, 0)),
        ],
        out_specs=(
            pl.BlockSpec((2, _TB), lambda i: (0, i)),
            pl.BlockSpec((2, _TB), lambda i: (0, i)),
        ),
        out_shape=(
            jax.ShapeDtypeStruct((2, T), jnp.float32),
            jax.ShapeDtypeStruct((2, T), jnp.int32),
        ),
    )(W, hs)
    return (w2t.T, e2t.T)


# speedup vs baseline: 1.1836x; 1.1836x over previous
import jax
import jax.numpy as jnp
from jax import lax
from jax.experimental import pallas as pl

_TB = 1024
_E = 16
_NEG_INF = float("-inf")


def _body(w_ref, hs_ref, wout_ref, eout_ref):
    lg = jax.lax.dot_general(
        w_ref[...], hs_ref[...], (((1,), (1,)), ((), ())),
        preferred_element_type=jnp.float32,
    )
    row = lax.broadcasted_iota(jnp.int32, lg.shape, 0)
    m1 = jnp.max(lg, axis=0, keepdims=True)
    i1 = jnp.min(jnp.where(lg == m1, row, _E), axis=0, keepdims=True)
    masked = jnp.where(row == i1, _NEG_INF, lg)
    m2 = jnp.max(masked, axis=0, keepdims=True)
    i2 = jnp.min(jnp.where(masked == m2, row, _E), axis=0, keepdims=True)
    t = jnp.exp(m2 - m1)
    denom = 1.0 + t
    wout_ref[...] = jnp.concatenate([1.0 / denom, t / denom], axis=0)
    eout_ref[...] = jnp.concatenate([i1, i2], axis=0)


@jax.jit
def kernel(hidden_states, W):
    hs = hidden_states.reshape(-1, hidden_states.shape[-1])
    T, d = hs.shape
    w2t, e2t = pl.pallas_call(
        _body,
        grid=(T // _TB,),
        in_specs=[
            pl.BlockSpec((_E, d), lambda i: (0, 0)),
            pl.BlockSpec((_TB, d), lambda i: (i, 0)),
        ],
        out_specs=(
            pl.BlockSpec((2, _TB), lambda i: (0, i)),
            pl.BlockSpec((2, _TB), lambda i: (0, i)),
        ),
        out_shape=(
            jax.ShapeDtypeStruct((2, T), jnp.float32),
            jax.ShapeDtypeStruct((2, T), jnp.int32),
        ),
    )(W, hs)
    return (w2t.T, e2t.T)
